# baseline (speedup 1.0000x reference)
import functools

import jax
import jax.numpy as jnp
from jax.experimental import pallas as pl
from jax.experimental.pallas import tpu as pltpu

_B, _S, _D = 4, 2048, 1024
_P, _L, _K = 64, 8, 2
_RBLK = 2048
_NBLK = _B * _S // _RBLK
_BLK_PER_B = _S // _RBLK


def _body(x_ref, keys_ref, prompts_ref, sim_ref, sep_ref, vals_ref, bp_ref,
          acc_ref, sem):
    i = pl.program_id(0)

    partial = jnp.sum(x_ref[...], axis=0, keepdims=True)
    acc_ref[pl.ds(i, 1), :] = partial

    @pl.when(i == _NBLK - 1)
    def _tail():
        xm = acc_ref[0:_B, :] * (1.0 / _S)
        xn = xm * jax.lax.rsqrt(
            jnp.maximum(jnp.sum(xm * xm, axis=1, keepdims=True), 1e-12))
        k = keys_ref[...]
        kn = k * jax.lax.rsqrt(
            jnp.maximum(jnp.sum(k * k, axis=1, keepdims=True), 1e-12))
        sim = jax.lax.dot_general(xn, kn, (((1,), (1,)), ((), ())),
                                  preferred_element_type=jnp.float32)
        sim_ref[...] = sim
        gram = jax.lax.dot_general(kn, kn, (((1,), (1,)), ((), ())),
                                   preferred_element_type=jnp.float32)
        r = jax.lax.broadcasted_iota(jnp.int32, (_P, _P), 0)
        c = jax.lax.broadcasted_iota(jnp.int32, (_P, _P), 1)
        diff = gram - (r == c).astype(jnp.float32)
        sep_ref[...] = (jnp.sum(diff * diff) * (1.0 / (_P * _P))).reshape(1, 1)

        col = jax.lax.broadcasted_iota(jnp.int32, (_B, _P), 1)
        v1 = jnp.max(sim, axis=1, keepdims=True)
        i1 = jnp.min(jnp.where(sim == v1, col, _P), axis=1, keepdims=True)
        masked = jnp.where(col == i1, -jnp.inf, sim)
        v2 = jnp.max(masked, axis=1, keepdims=True)
        i2 = jnp.min(jnp.where(masked == v2, col, _P), axis=1, keepdims=True)
        vals_ref[...] = jnp.concatenate([v1, v2], axis=1)

        copies = []
        for bb in range(_B):
            for kk in range(_K):
                idx = (i1 if kk == 0 else i2)[bb, 0]
                copies.append(pltpu.make_async_copy(
                    prompts_ref.at[idx],
                    bp_ref.at[bb, pl.ds(kk * _L, _L), :],
                    sem))
        for cp in copies:
            cp.start()
        for cp in copies:
            cp.wait()


def kernel(x_embed, prompt_keys, prompts, layer_idx):
    x_flat = x_embed.reshape(_B * _S, _D)
    sim, sep, vals, bp = pl.pallas_call(
        _body,
        grid=(_NBLK,),
        in_specs=[
            pl.BlockSpec((_RBLK, _D), lambda i: (i, 0)),
            pl.BlockSpec((_P, _D), lambda i: (0, 0)),
            pl.BlockSpec(memory_space=pltpu.MemorySpace.HBM),
        ],
        out_specs=[
            pl.BlockSpec((_B, _P), lambda i: (0, 0)),
            pl.BlockSpec((1, 1), lambda i: (0, 0)),
            pl.BlockSpec((_B, _K), lambda i: (0, 0)),
            pl.BlockSpec((_B, _K * _L, _D), lambda i: (0, 0, 0)),
        ],
        out_shape=[
            jax.ShapeDtypeStruct((_B, _P), jnp.float32),
            jax.ShapeDtypeStruct((1, 1), jnp.float32),
            jax.ShapeDtypeStruct((_B, _K), jnp.float32),
            jax.ShapeDtypeStruct((_B, _K * _L, _D), jnp.float32),
        ],
        scratch_shapes=[pltpu.VMEM((8, _D), jnp.float32),
                        pltpu.SemaphoreType.DMA],
        compiler_params=pltpu.CompilerParams(
            dimension_semantics=("arbitrary",)),
    )(x_flat, prompt_keys, prompts)
    orth = jnp.zeros((), jnp.float32)
    return (sim, orth, sep.reshape(()), vals, bp)


# speedup vs baseline: 1.0281x; 1.0281x over previous
import jax
import jax.numpy as jnp
from jax.experimental import pallas as pl
from jax.experimental.pallas import tpu as pltpu

_B, _S, _D = 4, 2048, 1024
_P, _L, _K = 64, 8, 2
_RBLK = _S
_NBLK = _B * _S // _RBLK


def _body(x_ref, keys_ref, prompts_ref, sim_ref, sep_ref, vals_ref, bp_ref,
          acc_ref):
    i = pl.program_id(0)
    partial = jnp.sum(x_ref[...], axis=0, keepdims=True)
    acc_ref[pl.ds(i, 1), :] = partial

    @pl.when(i == _NBLK - 1)
    def _tail():
        xm = acc_ref[0:_B, :] * (1.0 / _S)
        xn = xm * jax.lax.rsqrt(
            jnp.maximum(jnp.sum(xm * xm, axis=1, keepdims=True), 1e-12))
        k = keys_ref[...]
        kn = k * jax.lax.rsqrt(
            jnp.maximum(jnp.sum(k * k, axis=1, keepdims=True), 1e-12))
        sim = jax.lax.dot_general(xn, kn, (((1,), (1,)), ((), ())),
                                  preferred_element_type=jnp.float32)
        sim_ref[...] = sim
        gram = jax.lax.dot_general(kn, kn, (((1,), (1,)), ((), ())),
                                   preferred_element_type=jnp.float32)
        r = jax.lax.broadcasted_iota(jnp.int32, (_P, _P), 0)
        c = jax.lax.broadcasted_iota(jnp.int32, (_P, _P), 1)
        diff = gram - (r == c).astype(jnp.float32)
        sep_ref[...] = (jnp.sum(diff * diff) * (1.0 / (_P * _P))).reshape(1, 1)

        col = jax.lax.broadcasted_iota(jnp.int32, (_B, _P), 1)
        v1 = jnp.max(sim, axis=1, keepdims=True)
        i1 = jnp.min(jnp.where(sim == v1, col, _P), axis=1, keepdims=True)
        masked = jnp.where(col == i1, -jnp.inf, sim)
        v2 = jnp.max(masked, axis=1, keepdims=True)
        i2 = jnp.min(jnp.where(masked == v2, col, _P), axis=1, keepdims=True)
        vals_ref[...] = jnp.concatenate([v1, v2], axis=1)

        for bb in range(_B):
            for kk in range(_K):
                idx = (i1 if kk == 0 else i2)[bb, 0]
                bp_ref[bb, kk * _L:(kk + 1) * _L, :] = prompts_ref[idx]


def kernel(x_embed, prompt_keys, prompts, layer_idx):
    x_flat = x_embed.reshape(_B * _S, _D)
    sim, sep, vals, bp = pl.pallas_call(
        _body,
        grid=(_NBLK,),
        in_specs=[
            pl.BlockSpec((_RBLK, _D), lambda i: (i, 0)),
            pl.BlockSpec((_P, _D), lambda i: (0, 0)),
            pl.BlockSpec((_P, _L, _D), lambda i: (0, 0, 0)),
        ],
        out_specs=[
            pl.BlockSpec((_B, _P), lambda i: (0, 0)),
            pl.BlockSpec((1, 1), lambda i: (0, 0)),
            pl.BlockSpec((_B, _K), lambda i: (0, 0)),
            pl.BlockSpec((_B, _K * _L, _D), lambda i: (0, 0, 0)),
        ],
        out_shape=[
            jax.ShapeDtypeStruct((_B, _P), jnp.float32),
            jax.ShapeDtypeStruct((1, 1), jnp.float32),
            jax.ShapeDtypeStruct((_B, _K), jnp.float32),
            jax.ShapeDtypeStruct((_B, _K * _L, _D), jnp.float32),
        ],
        scratch_shapes=[pltpu.VMEM((8, _D), jnp.float32)],
        compiler_params=pltpu.CompilerParams(
            dimension_semantics=("arbitrary",)),
    )(x_flat, prompt_keys, prompts)
    orth = jnp.zeros((), jnp.float32)
    return (sim, orth, sep.reshape(()), vals, bp)
